# baseline (speedup 1.0000x reference)
import functools

import jax
import jax.numpy as jnp
from jax import lax
from jax.experimental import pallas as pl
from jax.experimental.pallas import tpu as pltpu
from jax.experimental.pallas import tpu_sc as plsc

_NC = 2
_NS = 16
_NW = _NC * _NS
_L = 16


def kernel(x, pos_table):
    B, T, D = x.shape
    TPW = T // _NW
    CR = 4
    NCH = TPW // CR
    NVR = D // _L
    U = 32
    NJ = NVR // U

    xf = x.reshape(B * T, D)

    mesh = plsc.VectorSubcoreMesh(core_axis_name="c", subcore_axis_name="s")

    @functools.partial(
        pl.kernel,
        mesh=mesh,
        out_type=jax.ShapeDtypeStruct((B * T, D), jnp.float32),
        scratch_types=(
            [pltpu.VMEM((CR, D), jnp.float32)] * 6
            + [pltpu.SemaphoreType.DMA] * 10
        ),
    )
    def sc_add(x_hbm, pos_hbm, out_hbm,
               xb0, xb1, xb2, xb3, pb0, pb1,
               xs0, xs1, xs2, xs3, os0, os1, os2, os3, ps0, ps1):
        xb = [xb0, xb1, xb2, xb3]
        pb = [pb0, pb1]
        xs = [xs0, xs1, xs2, xs3]
        osem = [os0, os1, os2, os3]
        ps = [ps0, ps1]

        c = lax.axis_index("c")
        s = lax.axis_index("s")
        wid = s * _NC + c
        t0 = wid * TPW

        def xrow(ci, b):
            return b * T + t0 + ci * CR

        def load_x(ci, b):
            pltpu.make_async_copy(
                x_hbm.at[pl.ds(xrow(ci, b), CR)], xb[b], xs[b]).start()

        def wait_x(ci, b):
            pltpu.make_async_copy(
                x_hbm.at[pl.ds(xrow(ci, b), CR)], xb[b], xs[b]).wait()

        def load_pos(ci, par):
            pltpu.make_async_copy(
                pos_hbm.at[pl.ds(t0 + ci * CR, CR)], pb[par], ps[par]).start()

        def wait_pos(ci, par):
            pltpu.make_async_copy(
                pos_hbm.at[pl.ds(t0 + ci * CR, CR)], pb[par], ps[par]).wait()

        def store_out(ci, b):
            pltpu.make_async_copy(
                xb[b], out_hbm.at[pl.ds(xrow(ci, b), CR)], osem[b]).start()

        def wait_out(ci, b):
            pltpu.make_async_copy(
                xb[b], out_hbm.at[pl.ds(xrow(ci, b), CR)], osem[b]).wait()

        load_pos(0, 0)
        load_x(0, 0)
        load_x(0, 1)

        def phase(ci, cis, b):
            tb = (b + 2) % 4
            if b < 2:
                @pl.when(ci >= 1)
                def _():
                    wait_out(ci - 1, tb)
                load_x(ci, tb)
            else:
                wait_out(ci, tb)

                @pl.when(ci < NCH - 1)
                def _():
                    load_x(ci + 1, tb)
            if b == 0:
                wait_pos(ci, cis)
            if b == 1:
                @pl.when(ci < NCH - 1)
                def _():
                    load_pos(ci + 1, cis ^ 1)
            wait_x(ci, b)

            for r in range(CR):
                def jbody(j, carry, r=r):
                    off = j * (U * _L)
                    for u in range(U):
                        sl = pl.ds(off + u * _L, _L)
                        plsc.addupdate(xb[b].at[r, sl], pb[cis][r, sl])
                    return carry

                lax.fori_loop(0, NJ, jbody, 0)
            store_out(ci, b)

        def outer(g, carry):
            for cis in (0, 1):
                ci = g * 2 + cis
                for b in range(4):
                    phase(ci, cis, b)
            return carry

        lax.fori_loop(0, NCH // 2, outer, 0)
        wait_out(NCH - 1, 2)
        wait_out(NCH - 1, 3)

    out = sc_add(xf, pos_table)
    return out.reshape(B, T, D)


# speedup vs baseline: 1.0024x; 1.0024x over previous
import functools

import jax
import jax.numpy as jnp
from jax import lax
from jax.experimental import pallas as pl
from jax.experimental.pallas import tpu as pltpu
from jax.experimental.pallas import tpu_sc as plsc

_NC = 2
_NS = 16
_NW = _NC * _NS
_L = 16


def kernel(x, pos_table):
    B, T, D = x.shape
    TPW = T // _NW
    CR = 4
    NCH = TPW // CR
    NVR = D // _L
    U = 16
    NJ = NVR // U

    xf = x.reshape(B * T, D)

    mesh = plsc.VectorSubcoreMesh(core_axis_name="c", subcore_axis_name="s")

    @functools.partial(
        pl.kernel,
        mesh=mesh,
        out_type=jax.ShapeDtypeStruct((B * T, D), jnp.float32),
        scratch_types=(
            [pltpu.VMEM((CR, D), jnp.float32)] * 6
            + [pltpu.SemaphoreType.DMA] * 10
        ),
    )
    def sc_add(x_hbm, pos_hbm, out_hbm,
               xb0, xb1, xb2, xb3, pb0, pb1,
               xs0, xs1, xs2, xs3, os0, os1, os2, os3, ps0, ps1):
        xb = [xb0, xb1, xb2, xb3]
        pb = [pb0, pb1]
        xs = [xs0, xs1, xs2, xs3]
        osem = [os0, os1, os2, os3]
        ps = [ps0, ps1]

        c = lax.axis_index("c")
        s = lax.axis_index("s")
        wid = s * _NC + c
        t0 = wid * TPW

        def xrow(ci, b):
            return b * T + t0 + ci * CR

        def load_x(ci, b):
            pltpu.make_async_copy(
                x_hbm.at[pl.ds(xrow(ci, b), CR)], xb[b], xs[b]).start()

        def wait_x(ci, b):
            pltpu.make_async_copy(
                x_hbm.at[pl.ds(xrow(ci, b), CR)], xb[b], xs[b]).wait()

        def load_pos(ci, par):
            pltpu.make_async_copy(
                pos_hbm.at[pl.ds(t0 + ci * CR, CR)], pb[par], ps[par]).start()

        def wait_pos(ci, par):
            pltpu.make_async_copy(
                pos_hbm.at[pl.ds(t0 + ci * CR, CR)], pb[par], ps[par]).wait()

        def store_out(ci, b):
            pltpu.make_async_copy(
                xb[b], out_hbm.at[pl.ds(xrow(ci, b), CR)], osem[b]).start()

        def wait_out(ci, b):
            pltpu.make_async_copy(
                xb[b], out_hbm.at[pl.ds(xrow(ci, b), CR)], osem[b]).wait()

        load_pos(0, 0)
        load_x(0, 0)

        def phase(ci, cis, b):
            tb = (b + 1) % 4
            if b < 3:
                @pl.when(ci >= 1)
                def _():
                    wait_out(ci - 1, tb)
                load_x(ci, tb)
            else:
                wait_out(ci, tb)

                @pl.when(ci < NCH - 1)
                def _():
                    load_x(ci + 1, tb)
            if b == 0:
                wait_pos(ci, cis)
            if b == 1:
                @pl.when(ci < NCH - 1)
                def _():
                    load_pos(ci + 1, cis ^ 1)
            wait_x(ci, b)

            for r in range(CR):
                def jbody(j, carry, r=r):
                    off = j * (U * _L)
                    for u in range(U):
                        sl = pl.ds(off + u * _L, _L)
                        plsc.addupdate(xb[b].at[r, sl], pb[cis][r, sl])
                    return carry

                lax.fori_loop(0, NJ, jbody, 0)
            store_out(ci, b)

        def outer(g, carry):
            for cis in (0, 1):
                ci = g * 2 + cis
                for b in range(4):
                    phase(ci, cis, b)
            return carry

        lax.fori_loop(0, NCH // 2, outer, 0)
        wait_out(NCH - 1, 1)
        wait_out(NCH - 1, 2)
        wait_out(NCH - 1, 3)

    out = sc_add(xf, pos_table)
    return out.reshape(B, T, D)


# speedup vs baseline: 1.0602x; 1.0576x over previous
import functools

import jax
import jax.numpy as jnp
from jax import lax
from jax.experimental import pallas as pl
from jax.experimental.pallas import tpu as pltpu
from jax.experimental.pallas import tpu_sc as plsc

_NC = 2
_NS = 16
_NW = _NC * _NS
_L = 16


def kernel(x, pos_table):
    B, T, D = x.shape
    TPW = T // _NW
    CR = 4
    NCH = TPW // CR
    NVR = D // _L
    U = 16
    NJ = NVR // U

    xf = x.reshape(B * T, D)

    mesh = plsc.VectorSubcoreMesh(core_axis_name="c", subcore_axis_name="s")

    @functools.partial(
        pl.kernel,
        mesh=mesh,
        out_type=jax.ShapeDtypeStruct((B * T, D), jnp.float32),
        scratch_types=(
            [pltpu.VMEM((CR, D), jnp.float32)] * 6
            + [pltpu.SemaphoreType.DMA] * 10
        ),
    )
    def sc_add(x_hbm, pos_hbm, out_hbm,
               xb0, xb1, xb2, xb3, pb0, pb1,
               xs0, xs1, xs2, xs3, os0, os1, os2, os3, ps0, ps1):
        xb = [xb0, xb1, xb2, xb3]
        pb = [pb0, pb1]
        xs = [xs0, xs1, xs2, xs3]
        osem = [os0, os1, os2, os3]
        ps = [ps0, ps1]

        c = lax.axis_index("c")
        s = lax.axis_index("s")
        wid = s * _NC + c
        t0 = wid * TPW

        def xrow(ci, b):
            return b * T + t0 + ci * CR

        def load_x(ci, b):
            pltpu.make_async_copy(
                x_hbm.at[pl.ds(xrow(ci, b), CR)], xb[b], xs[b]).start()

        def wait_x(ci, b):
            pltpu.make_async_copy(
                x_hbm.at[pl.ds(xrow(ci, b), CR)], xb[b], xs[b]).wait()

        def load_pos(ci, par):
            pltpu.make_async_copy(
                pos_hbm.at[pl.ds(t0 + ci * CR, CR)], pb[par], ps[par]).start()

        def wait_pos(ci, par):
            pltpu.make_async_copy(
                pos_hbm.at[pl.ds(t0 + ci * CR, CR)], pb[par], ps[par]).wait()

        def store_out(ci, b):
            pltpu.make_async_copy(
                xb[b], out_hbm.at[pl.ds(xrow(ci, b), CR)], osem[b]).start()

        def wait_out(ci, b):
            pltpu.make_async_copy(
                xb[b], out_hbm.at[pl.ds(xrow(ci, b), CR)], osem[b]).wait()

        def store_half(ci, b, half):
            pltpu.make_async_copy(
                xb[b].at[pl.ds(half * (CR // 2), CR // 2)],
                out_hbm.at[pl.ds(xrow(ci, b) + half * (CR // 2), CR // 2)],
                osem[b]).start()

        def wait_half(ci, b, half):
            pltpu.make_async_copy(
                xb[b].at[pl.ds(half * (CR // 2), CR // 2)],
                out_hbm.at[pl.ds(xrow(ci, b) + half * (CR // 2), CR // 2)],
                osem[b]).wait()

        def wait_out2(ci, b):
            wait_half(ci, b, 0)
            wait_half(ci, b, 1)

        load_pos(0, 0)
        load_x(0, 0)
        load_x(0, 1)

        def phase(ci, cis, b):
            tb = (b + 2) % 4
            if b < 2:
                @pl.when(ci >= 1)
                def _():
                    wait_out2(ci - 1, tb)
                load_x(ci, tb)
            else:
                wait_out2(ci, tb)

                @pl.when(ci < NCH - 1)
                def _():
                    load_x(ci + 1, tb)
            if b == 0:
                wait_pos(ci, cis)
            if b == 1:
                @pl.when(ci < NCH - 1)
                def _():
                    load_pos(ci + 1, cis ^ 1)
            wait_x(ci, b)

            for half in (0, 1):
                for r in range(half * (CR // 2), (half + 1) * (CR // 2)):
                    def jbody(j, carry, r=r):
                        off = j * (U * _L)
                        for u in range(U):
                            sl = pl.ds(off + u * _L, _L)
                            plsc.addupdate(xb[b].at[r, sl], pb[cis][r, sl])
                        return carry

                    lax.fori_loop(0, NJ, jbody, 0)
                store_half(ci, b, half)

        def outer(g, carry):
            for cis in (0, 1):
                ci = g * 2 + cis
                for b in range(4):
                    phase(ci, cis, b)
            return carry

        lax.fori_loop(0, NCH // 2, outer, 0)
        wait_out2(NCH - 1, 2)
        wait_out2(NCH - 1, 3)

    out = sc_add(xf, pos_table)
    return out.reshape(B, T, D)


# speedup vs baseline: 1.1009x; 1.0384x over previous
import functools

import jax
import jax.numpy as jnp
from jax import lax
from jax.experimental import pallas as pl
from jax.experimental.pallas import tpu as pltpu
from jax.experimental.pallas import tpu_sc as plsc

_NC = 2
_NS = 16
_NW = _NC * _NS
_L = 16


def kernel(x, pos_table):
    B, T, D = x.shape
    TPW = T // _NW
    CR = 2
    NCH = TPW // CR
    NVR = D // _L
    U = 16
    NJ = NVR // U

    xf = x.reshape(B * T, D)

    mesh = plsc.VectorSubcoreMesh(core_axis_name="c", subcore_axis_name="s")

    @functools.partial(
        pl.kernel,
        mesh=mesh,
        out_type=jax.ShapeDtypeStruct((B * T, D), jnp.float32),
        scratch_types=(
            [pltpu.VMEM((CR, D), jnp.float32)] * 10
            + [pltpu.SemaphoreType.DMA] * 18
        ),
    )
    def sc_add(x_hbm, pos_hbm, out_hbm,
               xb0, xb1, xb2, xb3, xb4, xb5, xb6, xb7, pb0, pb1,
               xs0, xs1, xs2, xs3, xs4, xs5, xs6, xs7,
               os0, os1, os2, os3, os4, os5, os6, os7, ps0, ps1):
        xb = [xb0, xb1, xb2, xb3, xb4, xb5, xb6, xb7]
        pb = [pb0, pb1]
        xs = [xs0, xs1, xs2, xs3, xs4, xs5, xs6, xs7]
        osem = [os0, os1, os2, os3, os4, os5, os6, os7]
        ps = [ps0, ps1]

        c = lax.axis_index("c")
        s = lax.axis_index("s")
        wid = s * _NC + c
        t0 = wid * TPW

        def xrow(ci, b):
            return b * T + t0 + ci * CR

        def load_x(ci, b, m):
            pltpu.make_async_copy(
                x_hbm.at[pl.ds(xrow(ci, b), CR)], xb[m], xs[m]).start()

        def wait_x(ci, b, m):
            pltpu.make_async_copy(
                x_hbm.at[pl.ds(xrow(ci, b), CR)], xb[m], xs[m]).wait()

        def load_pos(ci, par):
            pltpu.make_async_copy(
                pos_hbm.at[pl.ds(t0 + ci * CR, CR)], pb[par], ps[par]).start()

        def wait_pos(ci, par):
            pltpu.make_async_copy(
                pos_hbm.at[pl.ds(t0 + ci * CR, CR)], pb[par], ps[par]).wait()

        def store_out(ci, b, m):
            pltpu.make_async_copy(
                xb[m], out_hbm.at[pl.ds(xrow(ci, b), CR)], osem[m]).start()

        def wait_out(ci, b, m):
            pltpu.make_async_copy(
                xb[m], out_hbm.at[pl.ds(xrow(ci, b), CR)], osem[m]).wait()

        load_pos(0, 0)
        for b in range(4):
            load_x(0, b, b)

        def phase(ci, cis, b):
            mb = cis * 4 + b
            tb = (cis ^ 1) * 4 + b

            @pl.when(ci >= 1)
            def _():
                wait_out(ci - 1, b, tb)

            @pl.when(ci < NCH - 1)
            def _():
                load_x(ci + 1, b, tb)

            if b == 0:
                wait_pos(ci, cis)
            if b == 1:
                @pl.when(ci < NCH - 1)
                def _():
                    load_pos(ci + 1, cis ^ 1)
            wait_x(ci, b, mb)

            for r in range(CR):
                def jbody(j, carry, r=r):
                    off = j * (U * _L)
                    for u in range(U):
                        sl = pl.ds(off + u * _L, _L)
                        plsc.addupdate(xb[mb].at[r, sl], pb[cis][r, sl])
                    return carry

                lax.fori_loop(0, NJ, jbody, 0)
            store_out(ci, b, mb)

        def outer(g, carry):
            for cis in (0, 1):
                ci = g * 2 + cis
                for b in range(4):
                    phase(ci, cis, b)
            return carry

        lax.fori_loop(0, NCH // 2, outer, 0)
        for b in range(4):
            wait_out(NCH - 1, b, 4 + b)

    out = sc_add(xf, pos_table)
    return out.reshape(B, T, D)
